# Initial kernel scaffold; baseline (speedup 1.0000x reference)
import jax
import jax.numpy as jnp
from jax.experimental import pallas as pl


def kernel(x, edge_index, eps, W0, b0, W1, b1, W2, b2, fcW1, fcb1, fcW2, fcb2):
    raise NotImplementedError("write your pallas kernel here")


# speedup vs baseline: 6.7520x; 6.7520x over previous
import jax
import jax.numpy as jnp
from jax import lax
from jax.experimental import pallas as pl
from jax.experimental.pallas import tpu as pltpu
from jax.experimental.pallas import tpu_sc as plsc

N = 10000
NP = 10240
E = 320000
D = 128
C = 10
QW = D // 4
NSUB = 16
ROWS_PER = NP // NSUB
IDXW = 128
RPC = 4
IROWS_PER = 160
IROWS = IROWS_PER * NSUB
EPAD = IROWS * IDXW
NCHUNK = IROWS_PER // RPC


def _sc_body(h4_hbm, src_hbm, dst_hbm, zer_hbm, out_hbm,
             h_sh, p_sh, srcv, dstv, rows_v,
             isem0, isem1, gsem0, gsem1, ssem0, ssem1):
    cid = lax.axis_index("c")
    sid = lax.axis_index("s")
    rb = sid * ROWS_PER
    irb = sid * IROWS_PER
    isem = (isem0, isem1)
    gsem = (gsem0, gsem1)
    ssem = (ssem0, ssem1)

    def fire_idx(c, p):
        base = irb + c * RPC
        pltpu.async_copy(src_hbm.at[pl.ds(base, RPC)], srcv.at[p], isem[p])
        pltpu.async_copy(dst_hbm.at[pl.ds(base, RPC)], dstv.at[p], isem[p])

    def drain_idx(p):
        pltpu.make_async_copy(src_hbm.at[pl.ds(0, RPC)], srcv.at[p],
                              isem[p]).wait()
        pltpu.make_async_copy(dst_hbm.at[pl.ds(0, RPC)], dstv.at[p],
                              isem[p]).wait()

    def fire_gather(p):
        for j in range(RPC):
            pltpu.async_copy(h_sh.at[srcv.at[p, j]], rows_v.at[p, j], gsem[p])

    def drain_gather(p):
        for j in range(RPC):
            pltpu.make_async_copy(h4_hbm.at[0, pl.ds(0, IDXW)],
                                  rows_v.at[p, j], gsem[p]).wait()

    def fire_scatter(p):
        for j in range(RPC):
            pltpu.async_copy(rows_v.at[p, j], p_sh.at[dstv.at[p, j]], ssem[p],
                             add=True)

    def drain_scatter(p):
        for j in range(RPC):
            pltpu.make_async_copy(h4_hbm.at[0, pl.ds(0, IDXW)],
                                  rows_v.at[p, j], ssem[p]).wait()

    for t in range(2):
        q = 2 * cid + t
        pltpu.sync_copy(h4_hbm.at[q, pl.ds(rb, ROWS_PER)],
                        h_sh.at[pl.ds(rb, ROWS_PER)])
        pltpu.sync_copy(zer_hbm.at[pl.ds(rb, ROWS_PER)],
                        p_sh.at[pl.ds(rb, ROWS_PER)])
        plsc.subcore_barrier()

        fire_idx(0, 0)
        fire_idx(1, 1)
        drain_idx(0)
        fire_gather(0)
        drain_idx(1)
        fire_gather(1)
        drain_gather(0)
        fire_scatter(0)

        def pair(cp, carry):
            for b in range(2):
                c = 2 * cp + b
                p, o = b, 1 - b
                drain_scatter(p)
                fire_idx(c, p)
                drain_gather(o)
                fire_scatter(o)
                drain_idx(p)
                fire_gather(p)
            return carry

        lax.fori_loop(1, NCHUNK // 2, pair, 0)

        drain_gather(1)
        fire_scatter(1)
        drain_scatter(0)
        drain_scatter(1)
        plsc.subcore_barrier()
        pltpu.sync_copy(p_sh.at[pl.ds(rb, ROWS_PER)],
                        out_hbm.at[q, pl.ds(rb, ROWS_PER)])
        plsc.subcore_barrier()


def _sc_pooled(h4, src2, dst2, zer):
    mesh = plsc.VectorSubcoreMesh(core_axis_name="c", subcore_axis_name="s")
    kern = pl.kernel(
        _sc_body,
        out_type=jax.ShapeDtypeStruct((4, NP, QW), jnp.float32),
        mesh=mesh,
        scratch_types=[
            pltpu.VMEM_SHARED((NP, QW), jnp.float32),
            pltpu.VMEM_SHARED((NP, QW), jnp.float32),
            pltpu.VMEM((2, RPC, IDXW), jnp.int32),
            pltpu.VMEM((2, RPC, IDXW), jnp.int32),
            pltpu.VMEM((2, RPC, IDXW, QW), jnp.float32),
            pltpu.SemaphoreType.DMA,
            pltpu.SemaphoreType.DMA,
            pltpu.SemaphoreType.DMA,
            pltpu.SemaphoreType.DMA,
            pltpu.SemaphoreType.DMA,
            pltpu.SemaphoreType.DMA,
        ],
        compiler_params=pltpu.CompilerParams(use_tc_tiling_on_sc=False),
    )
    return kern(h4, src2, dst2, zer)


BLK = 2048


def _pad_mask():
    rows = pl.program_id(0) * BLK + lax.broadcasted_iota(jnp.int32, (BLK, 1), 0)
    return rows < N


def _dense(h_ref, p_ref, w_ref, b_ref, s_ref):
    h = jnp.concatenate([h_ref[k] for k in range(4)], axis=1)
    p = jnp.concatenate([p_ref[k] for k in range(4)], axis=1)
    z = s_ref[0, 0] * h + p
    z = jnp.dot(z, w_ref[...], preferred_element_type=jnp.float32) + b_ref[...]
    return jnp.where(_pad_mask(), jnp.maximum(z, 0.0), 0.0)


def _tc_body_mid(h_ref, p_ref, w_ref, b_ref, s_ref, o_ref):
    z = _dense(h_ref, p_ref, w_ref, b_ref, s_ref)
    for k in range(4):
        o_ref[k] = z[:, k * QW:(k + 1) * QW]


def _tc_body_last(h_ref, p_ref, w_ref, b_ref, s_ref, o_ref, g_ref):
    z = _dense(h_ref, p_ref, w_ref, b_ref, s_ref)
    for k in range(4):
        o_ref[k] = z[:, k * QW:(k + 1) * QW]

    @pl.when(pl.program_id(0) == 0)
    def _():
        g_ref[...] = jnp.zeros_like(g_ref)

    g_ref[...] += jnp.sum(z.reshape(BLK // 8, 8, D), axis=0)


def _tc_layer(h4, p4, W, b, scale, last):
    in_specs = [
        pl.BlockSpec((4, BLK, QW), lambda i: (0, i, 0)),
        pl.BlockSpec((4, BLK, QW), lambda i: (0, i, 0)),
        pl.BlockSpec((D, D), lambda i: (0, 0)),
        pl.BlockSpec((1, D), lambda i: (0, 0)),
        pl.BlockSpec(memory_space=pltpu.SMEM),
    ]
    if last:
        return pl.pallas_call(
            _tc_body_last,
            grid=(NP // BLK,),
            in_specs=in_specs,
            out_specs=[
                pl.BlockSpec((4, BLK, QW), lambda i: (0, i, 0)),
                pl.BlockSpec((8, D), lambda i: (0, 0)),
            ],
            out_shape=[
                jax.ShapeDtypeStruct((4, NP, QW), jnp.float32),
                jax.ShapeDtypeStruct((8, D), jnp.float32),
            ],
        )(h4, p4, W, b, scale)
    return pl.pallas_call(
        _tc_body_mid,
        grid=(NP // BLK,),
        in_specs=in_specs,
        out_specs=pl.BlockSpec((4, BLK, QW), lambda i: (0, i, 0)),
        out_shape=jax.ShapeDtypeStruct((4, NP, QW), jnp.float32),
    )(h4, p4, W, b, scale)


def _readout_body(g_ref, w1_ref, b1_ref, w2_ref, b2_ref, o_ref):
    g = jnp.sum(g_ref[...], axis=0, keepdims=True)
    t = jnp.dot(g, w1_ref[...], preferred_element_type=jnp.float32) + b1_ref[...]
    t = jnp.maximum(t, 0.0)
    z = jnp.dot(t, w2_ref[...], preferred_element_type=jnp.float32) + b2_ref[...]
    m = jnp.max(z, axis=1, keepdims=True)
    ez = jnp.exp(z - m)
    o_ref[...] = ez / jnp.sum(ez, axis=1, keepdims=True)


def _readout(g8, fcW1, fcb1, fcW2p, fcb2p):
    return pl.pallas_call(
        _readout_body,
        out_shape=jax.ShapeDtypeStruct((1, D), jnp.float32),
    )(g8, fcW1, fcb1, fcW2p, fcb2p)


def kernel(x, edge_index, eps, W0, b0, W1, b1, W2, b2, fcW1, fcb1, fcW2, fcb2):
    xp = jnp.pad(x, ((0, NP - N), (0, 0)))
    h4 = jnp.stack([xp[:, k * QW:(k + 1) * QW] for k in range(4)])
    src = edge_index[0]
    dst = edge_index[1]
    pad = EPAD - E
    src2 = jnp.concatenate([src, jnp.zeros((pad,), jnp.int32)]).reshape(IROWS, IDXW)
    dst2 = jnp.concatenate([dst, jnp.full((pad,), N, jnp.int32)]).reshape(IROWS, IDXW)
    zer = jnp.zeros((NP, QW), jnp.float32)

    g8 = None
    for i, (W, b) in enumerate(((W0, b0), (W1, b1), (W2, b2))):
        p4 = _sc_pooled(h4, src2, dst2, zer)
        scale = (1.0 + eps[i]).reshape(1, 1)
        if i == 2:
            h4, g8 = _tc_layer(h4, p4, W, b.reshape(1, D), scale, last=True)
        else:
            h4 = _tc_layer(h4, p4, W, b.reshape(1, D), scale, last=False)

    fcW2p = jnp.pad(fcW2, ((0, 0), (0, D - C)))
    fcb2p = jnp.concatenate([fcb2, jnp.full((D - C,), -1e30, jnp.float32)])
    out = _readout(g8, fcW1, fcb1.reshape(1, D), fcW2p, fcb2p.reshape(1, D))
    return out[0, :C]
